# baseline (speedup 1.0000x reference)
import jax
import jax.numpy as jnp
from jax import lax
from jax.experimental import pallas as pl
from jax.experimental.pallas import tpu as pltpu
from jax.experimental.pallas import tpu_sc as plsc

B = 16384
F = 26
FIELD_SIZE = 40000
NUM_WORKERS = 32
BPW = B // NUM_WORKERS
CHUNK = BPW * F
NVEC = BPW // 16
NSUB = 16


def _sc_body(x_hbm, w_hbm, out_hbm, ids_sh, acc_sh, s_v, idx_v, g_v, t_v, sem):
  cid = lax.axis_index("c")
  sid = lax.axis_index("s")
  wid = sid * 2 + cid
  base = wid * BPW

  pltpu.sync_copy(
      x_hbm.at[pl.ds(base * F, CHUNK)], ids_sh.at[pl.ds(sid * CHUNK, CHUNK)]
  )

  iota = lax.iota(jnp.int32, 16)

  def build_perm(f, _):
    perm_base = sid * CHUNK + f
    for v in range(NVEC):
      s_v[pl.ds(f * BPW + v * 16, 16)] = (v * 16 + iota) * F + perm_base
    return 0

  lax.fori_loop(0, F, build_perm, 0, unroll=False)

  pltpu.async_copy(ids_sh.at[s_v], idx_v, sem).wait()

  def add_off(f, _):
    off = f * FIELD_SIZE
    for v in range(NVEC):
      p0 = f * BPW + v * 16
      idx_v[pl.ds(p0, 16)] = idx_v[pl.ds(p0, 16)] + off
    return 0

  lax.fori_loop(0, F, add_off, 0, unroll=False)

  def build_tgt(v, _):
    tgt = sid * BPW + v * 16 + iota
    for f in range(F - 1):
      t_v[pl.ds(f * BPW + v * 16, 16)] = tgt
    return 0

  lax.fori_loop(0, NVEC, build_tgt, 0, unroll=False)

  pltpu.async_copy(w_hbm.at[idx_v], g_v, sem).wait()

  pltpu.sync_copy(g_v.at[pl.ds(0, BPW)], acc_sh.at[pl.ds(sid * BPW, BPW)])
  pltpu.sync_copy(g_v.at[pl.ds(BPW, (F - 1) * BPW)], acc_sh.at[t_v], add=True)
  pltpu.sync_copy(acc_sh.at[pl.ds(sid * BPW, BPW)], out_hbm.at[pl.ds(base, BPW)])


@jax.jit
def _sc_lookup(x_flat, w_flat):
  mesh = plsc.VectorSubcoreMesh(
      core_axis_name="c", subcore_axis_name="s", num_cores=2, num_subcores=16
  )
  return pl.kernel(
      _sc_body,
      out_type=jax.ShapeDtypeStruct((B,), jnp.float32),
      mesh=mesh,
      compiler_params=pltpu.CompilerParams(use_tc_tiling_on_sc=False),
      scratch_types=[
          pltpu.VMEM_SHARED((NSUB * CHUNK,), jnp.int32),
          pltpu.VMEM_SHARED((NSUB * BPW,), jnp.float32),
          pltpu.VMEM((CHUNK,), jnp.int32),
          pltpu.VMEM((CHUNK,), jnp.int32),
          pltpu.VMEM((CHUNK,), jnp.float32),
          pltpu.VMEM(((F - 1) * BPW,), jnp.int32),
          pltpu.SemaphoreType.DMA,
      ],
  )(x_flat, w_flat)


def kernel(x, fc_weight, bias):
  out = _sc_lookup(x.reshape(-1), fc_weight.reshape(-1))
  return out[:, None] + bias[None, :]


# speedup vs baseline: 1.1144x; 1.1144x over previous
import jax
import jax.numpy as jnp
from jax import lax
from jax.experimental import pallas as pl
from jax.experimental.pallas import tpu as pltpu
from jax.experimental.pallas import tpu_sc as plsc

B = 16384
F = 26
FIELD_SIZE = 40000
TABLE = F * FIELD_SIZE
NUM_WORKERS = 32
BPW = B // NUM_WORKERS
CHUNK = BPW * F
NVEC = BPW // 16
NSUB = 16
WSLICE = TABLE // NSUB


def _sc_body(x_hbm, w_hbm, b_hbm, out_hbm,
             w_sh, ids_sh, s_v, idx_v, g_v, o_v, b_v, sem, wsem):
  cid = lax.axis_index("c")
  sid = lax.axis_index("s")
  wid = sid * 2 + cid
  base = wid * BPW

  w_cp = pltpu.async_copy(
      w_hbm.at[pl.ds(sid * WSLICE, WSLICE)],
      w_sh.at[pl.ds(sid * WSLICE, WSLICE)],
      wsem,
  )

  pltpu.sync_copy(
      x_hbm.at[pl.ds(base * F, CHUNK)], ids_sh.at[pl.ds(sid * CHUNK, CHUNK)]
  )

  iota = lax.iota(jnp.int32, 16)

  def build_perm(f, _):
    perm_base = sid * CHUNK + f
    for v in range(NVEC):
      s_v[pl.ds(f * BPW + v * 16, 16)] = (v * 16 + iota) * F + perm_base
    return 0

  lax.fori_loop(0, F, build_perm, 0, unroll=False)

  pltpu.async_copy(ids_sh.at[s_v], idx_v, sem).wait()

  def add_off(f, _):
    off = f * FIELD_SIZE
    for v in range(NVEC):
      p0 = f * BPW + v * 16
      idx_v[pl.ds(p0, 16)] = idx_v[pl.ds(p0, 16)] + off
    return 0

  lax.fori_loop(0, F, add_off, 0, unroll=False)

  s_v[pl.ds(0, 16)] = iota * 0
  pltpu.async_copy(b_hbm.at[s_v.at[pl.ds(0, 16)]], b_v, sem).wait()
  bias_v = b_v[pl.ds(0, 16)]

  w_cp.wait()
  plsc.subcore_barrier()
  pltpu.async_copy(w_sh.at[idx_v], g_v, sem).wait()

  def reduce_vec(v, _):
    p0 = v * 16
    acc = bias_v + g_v[pl.ds(p0, 16)]
    for f in range(1, F):
      acc = acc + g_v[pl.ds(f * BPW + p0, 16)]
    o_v[pl.ds(p0, 16)] = acc
    return 0

  lax.fori_loop(0, NVEC, reduce_vec, 0, unroll=False)

  pltpu.sync_copy(o_v.at[pl.ds(0, BPW)], out_hbm.at[pl.ds(base, BPW)])


@jax.jit
def _sc_lookup(x_flat, w_flat, bias):
  mesh = plsc.VectorSubcoreMesh(
      core_axis_name="c", subcore_axis_name="s", num_cores=2, num_subcores=16
  )
  return pl.kernel(
      _sc_body,
      out_type=jax.ShapeDtypeStruct((B,), jnp.float32),
      mesh=mesh,
      compiler_params=pltpu.CompilerParams(use_tc_tiling_on_sc=False),
      scratch_types=[
          pltpu.VMEM_SHARED((TABLE,), jnp.float32),
          pltpu.VMEM_SHARED((NSUB * CHUNK,), jnp.int32),
          pltpu.VMEM((CHUNK,), jnp.int32),
          pltpu.VMEM((CHUNK,), jnp.int32),
          pltpu.VMEM((CHUNK,), jnp.float32),
          pltpu.VMEM((BPW,), jnp.float32),
          pltpu.VMEM((16,), jnp.float32),
          pltpu.SemaphoreType.DMA,
          pltpu.SemaphoreType.DMA,
      ],
  )(x_flat, w_flat, bias)


def kernel(x, fc_weight, bias):
  out = _sc_lookup(x.reshape(-1), fc_weight.reshape(-1), bias)
  return out[:, None]


# speedup vs baseline: 1.3321x; 1.1954x over previous
import jax
import jax.numpy as jnp
from jax import lax
from jax.experimental import pallas as pl
from jax.experimental.pallas import tpu as pltpu
from jax.experimental.pallas import tpu_sc as plsc

B = 16384
F = 26
FIELD_SIZE = 40000
TABLE = F * FIELD_SIZE
NUM_WORKERS = 32
BPW = B // NUM_WORKERS
CHUNK = BPW * F
NVEC = BPW // 16
NSUB = 16
WSLICE = TABLE // NSUB


def _sc_body(xt_hbm, wt_hbm, b_hbm, out_hbm,
             w_sh, idx_v, g_v, o_v, z_v, b_v, sem, wsem):
  cid = lax.axis_index("c")
  sid = lax.axis_index("s")
  wid = sid * 2 + cid
  base = wid * BPW

  w_cp = pltpu.async_copy(
      wt_hbm.at[0, pl.ds(sid * WSLICE, WSLICE)],
      w_sh.at[pl.ds(sid * WSLICE, WSLICE)],
      wsem,
  )

  id_cps = [
      pltpu.async_copy(
          xt_hbm.at[f, pl.ds(base, BPW)],
          idx_v.at[pl.ds(f * BPW, BPW)],
          sem,
      )
      for f in range(F)
  ]
  for cp in id_cps:
    cp.wait()

  iota = lax.iota(jnp.int32, 16)

  def add_off(f, _):
    off = f * FIELD_SIZE
    for v in range(NVEC):
      p0 = f * BPW + v * 16
      idx_v[pl.ds(p0, 16)] = idx_v[pl.ds(p0, 16)] + off
    return 0

  lax.fori_loop(0, F, add_off, 0, unroll=False)

  z_v[pl.ds(0, 16)] = iota * 0
  pltpu.async_copy(b_hbm.at[z_v.at[pl.ds(0, 16)]], b_v, sem).wait()
  bias_v = b_v[pl.ds(0, 16)]

  w_cp.wait()
  plsc.subcore_barrier()
  pltpu.async_copy(w_sh.at[idx_v], g_v, sem).wait()

  def reduce_vec(v, _):
    p0 = v * 16
    acc = bias_v + g_v[pl.ds(p0, 16)]
    for f in range(1, F):
      acc = acc + g_v[pl.ds(f * BPW + p0, 16)]
    o_v[pl.ds(p0, 16)] = acc
    return 0

  lax.fori_loop(0, NVEC, reduce_vec, 0, unroll=False)

  pltpu.sync_copy(o_v.at[pl.ds(0, BPW)], out_hbm.at[pl.ds(base, BPW)])


@jax.jit
def _sc_lookup(xt, wt, bias):
  mesh = plsc.VectorSubcoreMesh(
      core_axis_name="c", subcore_axis_name="s", num_cores=2, num_subcores=16
  )
  return pl.kernel(
      _sc_body,
      out_type=jax.ShapeDtypeStruct((B,), jnp.float32),
      mesh=mesh,
      compiler_params=pltpu.CompilerParams(use_tc_tiling_on_sc=False),
      scratch_types=[
          pltpu.VMEM_SHARED((TABLE,), jnp.float32),
          pltpu.VMEM((CHUNK,), jnp.int32),
          pltpu.VMEM((CHUNK,), jnp.float32),
          pltpu.VMEM((BPW,), jnp.float32),
          pltpu.VMEM((16,), jnp.int32),
          pltpu.VMEM((16,), jnp.float32),
          pltpu.SemaphoreType.DMA,
          pltpu.SemaphoreType.DMA,
      ],
  )(xt, wt, bias)


def kernel(x, fc_weight, bias):
  return _sc_lookup(x.T, fc_weight.T, bias)[:, None]
